# baseline (speedup 1.0000x reference)
import functools

import jax
import jax.numpy as jnp
from jax import lax
from jax.experimental import pallas as pl
from jax.experimental.pallas import tpu as pltpu
from jax.experimental.pallas import tpu_sc as plsc

NC = 2
NS = 16
NW = NC * NS
EP = 163840
EPW = EP // NW
CQ = 1024
NQ = EPW // CQ
EQ = EP // 4
PAD_ROWS = 16


def _tc_fold_body(x4_ref, b0_ref, b1_ref, b2_ref, b3_ref, k4_ref, o_ref):
    x4t = x4_ref[...].T.astype(jnp.bfloat16)
    tt4 = jnp.dot(k4_ref[...].astype(jnp.bfloat16), x4t,
                  preferred_element_type=jnp.float32)
    bts = (b0_ref, b1_ref, b2_ref, b3_ref)
    accs = []
    for g in range(4):
        bt = bts[g][...]
        a = tt4[g * 544 + 512:g * 544 + 544, :]
        for b in range(16):
            a = a + (bt[b:b + 1, :]
                     * tt4[g * 544 + b * 32:g * 544 + (b + 1) * 32, :])
        accs.append(a)
    o_ref[...] = jnp.concatenate(accs, axis=0).T


def _tc_add_body(p_ref, o_ref):
    o_ref[...] = p_ref[0] + p_ref[1]


def _sc_gather(atom, pt, ad):
    mesh = plsc.VectorSubcoreMesh(core_axis_name="c", subcore_axis_name="s")

    @functools.partial(
        pl.kernel,
        out_type=jax.ShapeDtypeStruct((EQ, 4, ad), jnp.float32),
        mesh=mesh,
        scratch_types=[pltpu.VMEM((CQ,), jnp.int32)] * NQ
        + [pltpu.VMEM((2, CQ, ad), jnp.float32)]
        + [pltpu.SemaphoreType.DMA] * 2,
        compiler_params=pltpu.CompilerParams(use_tc_tiling_on_sc=False),
    )
    def k(atom_hbm, pt_hbm, x_hbm, *rest):
        idx = rest[:NQ]
        rows_v = rest[NQ]
        sems = rest[NQ + 1:]
        cid = lax.axis_index("c")
        sid = lax.axis_index("s")
        wid = cid * NS + sid
        base = wid * EPW
        g = wid // 8
        rb = (wid % 8) * EPW
        for q in range(NQ):
            pltpu.sync_copy(pt_hbm.at[1, pl.ds(base + q * CQ, CQ)], idx[q])
        pltpu.async_copy(atom_hbm.at[idx[0]], rows_v.at[0], sems[0])
        for q in range(NQ):
            if q + 1 < NQ:
                pltpu.async_copy(atom_hbm.at[idx[q + 1]],
                                 rows_v.at[(q + 1) % 2], sems[(q + 1) % 2])
            pltpu.make_async_copy(atom_hbm.at[idx[q]],
                                  rows_v.at[q % 2], sems[q % 2]).wait()
            pltpu.sync_copy(rows_v.at[q % 2],
                            x_hbm.at[pl.ds(rb + q * CQ, CQ), g, :])

    return k(atom, pt)


def _sc_scatter(t, pt, zeros, n, ad):
    nacc = n + PAD_ROWS
    rpc = nacc // NS
    mesh = plsc.VectorSubcoreMesh(core_axis_name="c", subcore_axis_name="s")

    @functools.partial(
        pl.kernel,
        out_type=jax.ShapeDtypeStruct((NC, nacc, ad), jnp.float32),
        mesh=mesh,
        scratch_types=[pltpu.VMEM((CQ,), jnp.int32)] * NQ
        + [pltpu.VMEM((2, CQ, ad), jnp.float32)]
        + [pltpu.VMEM_SHARED((nacc, ad), jnp.float32)]
        + [pltpu.SemaphoreType.DMA] * 2,
        compiler_params=pltpu.CompilerParams(use_tc_tiling_on_sc=False),
    )
    def k(t_hbm, pt_hbm, z_hbm, out_hbm, *rest):
        idx = rest[:NQ]
        rows_v = rest[NQ]
        acc_sh = rest[NQ + 1]
        sems = rest[NQ + 2:]
        cid = lax.axis_index("c")
        sid = lax.axis_index("s")
        wid = cid * NS + sid
        base = wid * EPW
        g = wid // 8
        rb = (wid % 8) * EPW
        pltpu.sync_copy(z_hbm.at[pl.ds(sid * rpc, rpc)],
                        acc_sh.at[pl.ds(sid * rpc, rpc)])
        for q in range(NQ):
            pltpu.sync_copy(pt_hbm.at[0, pl.ds(base + q * CQ, CQ)], idx[q])
        plsc.subcore_barrier()
        pltpu.async_copy(t_hbm.at[pl.ds(rb, CQ), g, :], rows_v.at[0], sems[0])
        for q in range(NQ):
            if q + 1 < NQ:
                pltpu.async_copy(t_hbm.at[pl.ds(rb + (q + 1) * CQ, CQ), g, :],
                                 rows_v.at[(q + 1) % 2], sems[(q + 1) % 2])
            pltpu.make_async_copy(t_hbm.at[pl.ds(rb + q * CQ, CQ), g, :],
                                  rows_v.at[q % 2], sems[q % 2]).wait()
            pltpu.sync_copy(rows_v.at[q % 2], acc_sh.at[idx[q]], add=True)
        plsc.subcore_barrier()
        pltpu.sync_copy(acc_sh.at[pl.ds(sid * rpc, rpc)],
                        out_hbm.at[cid, pl.ds(sid * rpc, rpc)])

    return k(t, pt, zeros)


def kernel(atom_features, bond_features, pair_indices, kernel, bias):
    n, ad = atom_features.shape
    e, bd = bond_features.shape
    pad = EP - e
    nacc = n + PAD_ROWS
    assert nacc % NS == 0 and EQ % 128 == 0 and pad >= 0

    kr = kernel.reshape(bd, ad, ad)
    k2 = kr.transpose(2, 0, 1).reshape(ad, bd * ad)
    b2 = bias.reshape(ad, ad).T
    k2t = jnp.concatenate([k2, b2], axis=1).T
    k4 = jnp.kron(jnp.eye(4, dtype=jnp.float32), k2t)

    pt = pair_indices.astype(jnp.int32).T
    padcols = jnp.stack([jnp.full((pad,), n, jnp.int32),
                         jnp.zeros((pad,), jnp.int32)])
    pt_p = jnp.concatenate([pt, padcols], axis=1)

    btp = jnp.concatenate(
        [bond_features.T, jnp.zeros((bd, pad), jnp.float32)], axis=1)
    btg = [btp[:, g * EQ:(g + 1) * EQ] for g in range(4)]

    x = _sc_gather(atom_features, pt_p, ad)
    x4 = x.reshape(EQ, 4 * ad)

    be4 = 2048
    t4 = pl.pallas_call(
        _tc_fold_body,
        grid=(EQ // be4,),
        in_specs=[pl.BlockSpec((be4, 4 * ad), lambda i: (i, 0))]
        + [pl.BlockSpec((bd, be4), lambda i: (0, i))] * 4
        + [pl.BlockSpec((4 * (bd + 1) * ad, 4 * ad), lambda i: (0, 0))],
        out_specs=pl.BlockSpec((be4, 4 * ad), lambda i: (i, 0)),
        out_shape=jax.ShapeDtypeStruct((EQ, 4 * ad), jnp.float32),
    )(x4, *btg, k4)
    transformed = t4.reshape(EQ, 4, ad)

    zeros = jnp.zeros((nacc, ad), jnp.float32)
    partials = _sc_scatter(transformed, pt_p, zeros, n, ad)

    nb = nacc * ad // 128
    p4 = partials.reshape(NC, nb, 128)
    out4 = pl.pallas_call(
        _tc_add_body,
        grid=(1,),
        in_specs=[pl.BlockSpec((NC, nb, 128), lambda i: (0, 0, 0))],
        out_specs=pl.BlockSpec((nb, 128), lambda i: (0, 0)),
        out_shape=jax.ShapeDtypeStruct((nb, 128), jnp.float32),
    )(p4)
    return out4.reshape(nacc, ad)[:n]


# speedup vs baseline: 1.8278x; 1.8278x over previous
import functools

import jax
import jax.numpy as jnp
from jax import lax
from jax.experimental import pallas as pl
from jax.experimental.pallas import tpu as pltpu
from jax.experimental.pallas import tpu_sc as plsc

NC = 2
NS = 16
NW = NC * NS
CQ = 1000


def _tc_fold_body(x4_ref, bt4_ref, k4_ref, o_ref):
    x4t = x4_ref[...].T.astype(jnp.bfloat16)
    tt4 = jnp.dot(k4_ref[...].astype(jnp.bfloat16), x4t,
                  preferred_element_type=jnp.float32)
    bt4 = bt4_ref[0]
    accs = []
    for g in range(4):
        a = tt4[g * 544 + 512:g * 544 + 544, :]
        for b in range(16):
            a = a + (bt4[g * 16 + b:g * 16 + b + 1, :]
                     * tt4[g * 544 + b * 32:g * 544 + (b + 1) * 32, :])
        accs.append(a)
    o_ref[...] = jnp.concatenate(accs, axis=0).T


def _tc_add_body(p_ref, o_ref):
    o_ref[...] = p_ref[0] + p_ref[1]


def _sc_gather(atom, pt, e, ad):
    epw = e // NW
    nq = epw // CQ
    mesh = plsc.VectorSubcoreMesh(core_axis_name="c", subcore_axis_name="s")

    @functools.partial(
        pl.kernel,
        out_type=jax.ShapeDtypeStruct((e, ad), jnp.float32),
        mesh=mesh,
        scratch_types=[pltpu.VMEM((CQ,), jnp.int32)] * nq
        + [pltpu.VMEM((2, CQ, ad), jnp.float32)]
        + [pltpu.SemaphoreType.DMA] * 2,
        compiler_params=pltpu.CompilerParams(use_tc_tiling_on_sc=False),
    )
    def k(atom_hbm, pt_hbm, x_hbm, *rest):
        idx = rest[:nq]
        rows_v = rest[nq]
        sems = rest[nq + 1:]
        cid = lax.axis_index("c")
        sid = lax.axis_index("s")
        wid = cid * NS + sid
        base = wid * epw
        for q in range(nq):
            pltpu.sync_copy(pt_hbm.at[1, pl.ds(base + q * CQ, CQ)], idx[q])
        pltpu.async_copy(atom_hbm.at[idx[0]], rows_v.at[0], sems[0])
        for q in range(nq):
            if q + 1 < nq:
                pltpu.async_copy(atom_hbm.at[idx[q + 1]],
                                 rows_v.at[(q + 1) % 2], sems[(q + 1) % 2])
            pltpu.make_async_copy(atom_hbm.at[idx[q]],
                                  rows_v.at[q % 2], sems[q % 2]).wait()
            pltpu.sync_copy(rows_v.at[q % 2],
                            x_hbm.at[pl.ds(base + q * CQ, CQ)])

    return k(atom, pt)


def _sc_scatter(t, pt, zeros, n, e, ad):
    epw = e // NW
    rpc = n // NS
    mesh = plsc.VectorSubcoreMesh(core_axis_name="c", subcore_axis_name="s")

    nq = epw // CQ

    @functools.partial(
        pl.kernel,
        out_type=jax.ShapeDtypeStruct((NC, n, ad), jnp.float32),
        mesh=mesh,
        scratch_types=[pltpu.VMEM((CQ,), jnp.int32)] * nq
        + [pltpu.VMEM((2, CQ, ad), jnp.float32)]
        + [pltpu.VMEM_SHARED((n, ad), jnp.float32)]
        + [pltpu.SemaphoreType.DMA] * 2,
        compiler_params=pltpu.CompilerParams(use_tc_tiling_on_sc=False),
    )
    def k(t_hbm, pt_hbm, z_hbm, out_hbm, *rest):
        idx = rest[:nq]
        rows_v = rest[nq]
        acc_sh = rest[nq + 1]
        sems = rest[nq + 2:]
        cid = lax.axis_index("c")
        sid = lax.axis_index("s")
        wid = cid * NS + sid
        base = wid * epw
        pltpu.sync_copy(z_hbm.at[pl.ds(sid * rpc, rpc)],
                        acc_sh.at[pl.ds(sid * rpc, rpc)])
        for q in range(nq):
            pltpu.sync_copy(pt_hbm.at[0, pl.ds(base + q * CQ, CQ)], idx[q])
        plsc.subcore_barrier()
        pltpu.async_copy(t_hbm.at[pl.ds(base, CQ)], rows_v.at[0], sems[0])
        for q in range(nq):
            if q + 1 < nq:
                pltpu.async_copy(t_hbm.at[pl.ds(base + (q + 1) * CQ, CQ)],
                                 rows_v.at[(q + 1) % 2], sems[(q + 1) % 2])
            pltpu.make_async_copy(t_hbm.at[pl.ds(base + q * CQ, CQ)],
                                  rows_v.at[q % 2], sems[q % 2]).wait()
            pltpu.sync_copy(rows_v.at[q % 2], acc_sh.at[idx[q]], add=True)
        plsc.subcore_barrier()
        pltpu.sync_copy(acc_sh.at[pl.ds(sid * rpc, rpc)],
                        out_hbm.at[cid, pl.ds(sid * rpc, rpc)])

    return k(t, pt, zeros)


def kernel(atom_features, bond_features, pair_indices, kernel, bias):
    n, ad = atom_features.shape
    e, bd = bond_features.shape
    assert e % (NW * CQ) == 0 and CQ % 8 == 0 and n % NS == 0

    kr = kernel.reshape(bd, ad, ad)
    k2 = kr.transpose(2, 0, 1).reshape(ad, bd * ad)
    b2 = bias.reshape(ad, ad).T
    k2t = jnp.concatenate([k2, b2], axis=1).T
    k4 = jnp.kron(jnp.eye(4, dtype=jnp.float32), k2t)

    pt = pair_indices.astype(jnp.int32).T

    x = _sc_gather(atom_features, pt, e, ad)

    be = 8000
    x4 = x.reshape(e // 4, 4 * ad)
    bt43 = bond_features.reshape(e // be, be // 4, 4 * bd).transpose(0, 2, 1)
    t4 = pl.pallas_call(
        _tc_fold_body,
        grid=(e // be,),
        in_specs=[
            pl.BlockSpec((be // 4, 4 * ad), lambda i: (i, 0)),
            pl.BlockSpec((1, 4 * bd, be // 4), lambda i: (i, 0, 0)),
            pl.BlockSpec((4 * (bd + 1) * ad, 4 * ad), lambda i: (0, 0)),
        ],
        out_specs=pl.BlockSpec((be // 4, 4 * ad), lambda i: (i, 0)),
        out_shape=jax.ShapeDtypeStruct((e // 4, 4 * ad), jnp.float32),
    )(x4, bt43, k4)
    transformed = t4.reshape(e, ad)

    zeros = jnp.zeros((n, ad), jnp.float32)
    partials = _sc_scatter(transformed, pt, zeros, n, e, ad)

    p4 = partials.reshape(NC, n * ad // 128, 128)
    nb = n * ad // 128
    out4 = pl.pallas_call(
        _tc_add_body,
        grid=(1,),
        in_specs=[pl.BlockSpec((NC, nb, 128), lambda i: (0, 0, 0))],
        out_specs=pl.BlockSpec((nb, 128), lambda i: (0, 0)),
        out_shape=jax.ShapeDtypeStruct((nb, 128), jnp.float32),
    )(p4)
    return out4.reshape(n, ad)
